# baseline (speedup 1.0000x reference)
import functools

import jax
import jax.numpy as jnp
from jax import lax
from jax.experimental import pallas as pl
from jax.experimental.pallas import tpu as pltpu
from jax.experimental.pallas import tpu_sc as plsc

INF = 100000.0
DIM = 96
RES = 32
NUM_PATCH = RES * RES
K = 10
BATCH = 32
RB = 512
NB = NUM_PATCH // RB
NCH = NUM_PATCH // 128

_SORT8 = [
    (0, 1), (2, 3), (4, 5), (6, 7),
    (0, 2), (1, 3), (4, 6), (5, 7),
    (1, 2), (5, 6),
    (0, 4), (1, 5), (2, 6), (3, 7),
    (2, 4), (3, 5),
    (1, 2), (3, 4), (5, 6),
]


def _norm_body(feat_ref, out_ref):
    x = feat_ref[0]
    nrm = jnp.sqrt(jnp.sum(x * x, axis=1, keepdims=True))
    out_ref[0] = x / jnp.clip(nrm, 1e-12, None)


def _bias_body(rel_ref, out_ref):
    r = pl.program_id(0)
    j0 = lax.broadcasted_iota(jnp.int32, (NUM_PATCH, RB), 0)
    i0 = r * RB + lax.broadcasted_iota(jnp.int32, (NUM_PATCH, RB), 1)
    nbr = ((jnp.abs((j0 >> 5) - (i0 >> 5)) <= 1)
           & (jnp.abs((j0 & 31) - (i0 & 31)) <= 1))
    out_ref[...] = rel_ref[0] + jnp.where(nbr, INF, 0.0) + 2.0


def _body(xn_ref, bias_ref, out_ref):
    r = pl.program_id(0)
    b = pl.program_id(1)
    xn = xn_ref[0]
    xr = xn_ref[0, pl.ds(r * RB, RB), :]
    prod = lax.dot_general(xn, xr, (((1,), (1,)), ((), ())),
                           preferred_element_type=jnp.float32)
    d = bias_ref[...] - 2.0 * prod
    ib = lax.bitcast_convert_type(d, jnp.int32)
    ch = []
    for c in range(NCH):
        sl = (ib[c * 128:(c + 1) * 128, :] & jnp.int32(-8)) | jnp.int32(c)
        ch.append(lax.bitcast_convert_type(sl, jnp.float32))
    ch.append(jnp.full((128, RB), jnp.inf, dtype=jnp.float32))

    for a, c in _SORT8:
        lo = jnp.minimum(ch[a], ch[c])
        hi = jnp.maximum(ch[a], ch[c])
        ch[a], ch[c] = lo, hi

    iota0 = lax.broadcasted_iota(jnp.int32, (128, RB), 0)
    outs = []
    for k in range(K):
        m = jnp.min(ch[0], axis=0)
        jm = jnp.min(jnp.where(ch[0] == m[None, :], iota0, jnp.int32(1 << 30)),
                     axis=0)
        mc = lax.bitcast_convert_type(m, jnp.int32) & 7
        outs.append(mc * 128 + jm + b * NUM_PATCH)
        upper = min(NCH, K - 1 - k)
        if upper > 0:
            sel = iota0 == jm[None, :]
            for lv in range(upper):
                ch[lv] = jnp.where(sel, ch[lv + 1], ch[lv])
    outs += [jnp.zeros((RB,), jnp.int32)] * (16 - K)
    out_ref[0] = jnp.stack(outs, axis=0)


@functools.partial(jax.jit, static_argnums=())
def _topk_call(node_feature, relative_pos):
    xn = pl.pallas_call(
        _norm_body,
        grid=(BATCH,),
        in_specs=[pl.BlockSpec((1, NUM_PATCH, DIM), lambda b: (b, 0, 0))],
        out_specs=pl.BlockSpec((1, NUM_PATCH, DIM), lambda b: (b, 0, 0)),
        out_shape=jax.ShapeDtypeStruct((BATCH, NUM_PATCH, DIM), jnp.float32),
    )(node_feature)
    bias = pl.pallas_call(
        _bias_body,
        grid=(NB,),
        in_specs=[pl.BlockSpec((1, NUM_PATCH, RB), lambda r: (0, 0, r))],
        out_specs=pl.BlockSpec((NUM_PATCH, RB), lambda r: (0, r)),
        out_shape=jax.ShapeDtypeStruct((NUM_PATCH, NUM_PATCH), jnp.float32),
    )(relative_pos)
    return pl.pallas_call(
        _body,
        grid=(NB, BATCH),
        in_specs=[
            pl.BlockSpec((1, NUM_PATCH, DIM), lambda r, b: (b, 0, 0)),
            pl.BlockSpec((NUM_PATCH, RB), lambda r, b: (0, r)),
        ],
        out_specs=pl.BlockSpec((1, 16, RB), lambda r, b: (b, 0, r)),
        out_shape=jax.ShapeDtypeStruct((BATCH, 16, NUM_PATCH), jnp.int32),
    )(xn, bias)


TOTAL_E = BATCH * NUM_PATCH * K
NW = 32
EPT = TOTAL_E // NW


def _assemble_body(tk_hbm, out_hbm, tk_v, out_v):
    wid = lax.axis_index("s") * 2 + lax.axis_index("c")
    base = wid * EPT
    pltpu.sync_copy(tk_hbm.at[pl.ds(base, EPT)], tk_v)
    lane = lax.iota(jnp.int32, 16)
    zero = jnp.zeros((16,), jnp.int32)

    def body(i, carry):
        e_loc = i * 16 + lane
        dst = tk_v[pl.ds(i * 16, 16)]
        src = (base + e_loc) // K
        flat = e_loc * 3
        plsc.store_scatter(out_v, [flat], dst)
        plsc.store_scatter(out_v, [flat + 1], src)
        plsc.store_scatter(out_v, [flat + 2], zero)
        return carry

    lax.fori_loop(0, EPT // 16, body, 0)
    pltpu.sync_copy(out_v, out_hbm.at[pl.ds(base * 3, EPT * 3)])


@functools.cache
def _get_assemble():
    mesh = plsc.VectorSubcoreMesh(core_axis_name="c", subcore_axis_name="s")
    return pl.kernel(
        _assemble_body,
        mesh=mesh,
        out_type=jax.ShapeDtypeStruct((TOTAL_E * 3,), jnp.int32),
        scratch_types=[
            pltpu.VMEM((EPT,), jnp.int32),
            pltpu.VMEM((EPT * 3,), jnp.int32),
        ],
        compiler_params=pltpu.CompilerParams(needs_layout_passes=False,
                                             use_tc_tiling_on_sc=True),
    )


def kernel(node_feature, relative_pos):
    tk_raw = _topk_call(node_feature, relative_pos)
    tk = tk_raw.transpose(0, 2, 1)[:, :, :K]
    edge_list = _get_assemble()(tk.reshape(-1)).reshape(TOTAL_E, 3)
    return (edge_list, 1)


# speedup vs baseline: 1.8026x; 1.8026x over previous
import functools

import jax
import jax.numpy as jnp
from jax import lax
from jax.experimental import pallas as pl
from jax.experimental.pallas import tpu as pltpu
from jax.experimental.pallas import tpu_sc as plsc

INF = 100000.0
DIM = 96
RES = 32
NUM_PATCH = RES * RES
K = 10
BATCH = 32
RB = 512
NB = NUM_PATCH // RB
NCH = NUM_PATCH // 128

_SORT8 = [
    (0, 1), (2, 3), (4, 5), (6, 7),
    (0, 2), (1, 3), (4, 6), (5, 7),
    (1, 2), (5, 6),
    (0, 4), (1, 5), (2, 6), (3, 7),
    (2, 4), (3, 5),
    (1, 2), (3, 4), (5, 6),
]


def _norm_body(feat_ref, out_ref):
    x = feat_ref[0]
    nrm = jnp.sqrt(jnp.sum(x * x, axis=1, keepdims=True))
    out_ref[0] = x / jnp.clip(nrm, 1e-12, None)


def _bias_body(rel_ref, out_ref):
    r = pl.program_id(0)
    j0 = lax.broadcasted_iota(jnp.int32, (NUM_PATCH, RB), 0)
    i0 = r * RB + lax.broadcasted_iota(jnp.int32, (NUM_PATCH, RB), 1)
    nbr = ((jnp.abs((j0 >> 5) - (i0 >> 5)) <= 1)
           & (jnp.abs((j0 & 31) - (i0 & 31)) <= 1))
    out_ref[...] = rel_ref[0] + jnp.where(nbr, INF, 0.0) + 2.0


def _body(xn_ref, bias_ref, out_ref):
    r = pl.program_id(0)
    b = pl.program_id(1)
    xn = xn_ref[0]
    xr = xn_ref[0, pl.ds(r * RB, RB), :]
    prod = lax.dot_general(xn, xr, (((1,), (1,)), ((), ())),
                           preferred_element_type=jnp.float32)
    d = bias_ref[...] - 2.0 * prod
    ib = lax.bitcast_convert_type(d, jnp.int32)
    ch = []
    for c in range(NCH):
        sl = (ib[c * 128:(c + 1) * 128, :] & jnp.int32(-8)) | jnp.int32(c)
        ch.append(lax.bitcast_convert_type(sl, jnp.float32))
    ch.append(jnp.full((128, RB), jnp.inf, dtype=jnp.float32))

    for a, c in _SORT8:
        lo = jnp.minimum(ch[a], ch[c])
        hi = jnp.maximum(ch[a], ch[c])
        ch[a], ch[c] = lo, hi

    iota0 = lax.broadcasted_iota(jnp.int32, (128, RB), 0)
    outs = []
    for k in range(K):
        m = jnp.min(ch[0], axis=0)
        jm = jnp.min(jnp.where(ch[0] == m[None, :], iota0, jnp.int32(1 << 30)),
                     axis=0)
        mc = lax.bitcast_convert_type(m, jnp.int32) & 7
        outs.append(mc * 128 + jm + b * NUM_PATCH)
        upper = min(NCH, K - 1 - k)
        if upper > 0:
            sel = iota0 == jm[None, :]
            for lv in range(upper):
                ch[lv] = jnp.where(sel, ch[lv + 1], ch[lv])
    outs += [jnp.zeros((RB,), jnp.int32)] * (16 - K)
    out_ref[0] = jnp.stack(outs, axis=0)


@functools.partial(jax.jit, static_argnums=())
def _topk_call(node_feature, relative_pos):
    xn = pl.pallas_call(
        _norm_body,
        grid=(BATCH,),
        in_specs=[pl.BlockSpec((1, NUM_PATCH, DIM), lambda b: (b, 0, 0))],
        out_specs=pl.BlockSpec((1, NUM_PATCH, DIM), lambda b: (b, 0, 0)),
        out_shape=jax.ShapeDtypeStruct((BATCH, NUM_PATCH, DIM), jnp.float32),
    )(node_feature)
    bias = pl.pallas_call(
        _bias_body,
        grid=(NB,),
        in_specs=[pl.BlockSpec((1, NUM_PATCH, RB), lambda r: (0, 0, r))],
        out_specs=pl.BlockSpec((NUM_PATCH, RB), lambda r: (0, r)),
        out_shape=jax.ShapeDtypeStruct((NUM_PATCH, NUM_PATCH), jnp.float32),
    )(relative_pos)
    return pl.pallas_call(
        _body,
        grid=(NB, BATCH),
        in_specs=[
            pl.BlockSpec((1, NUM_PATCH, DIM), lambda r, b: (b, 0, 0)),
            pl.BlockSpec((NUM_PATCH, RB), lambda r, b: (0, r)),
        ],
        out_specs=pl.BlockSpec((1, 16, RB), lambda r, b: (b, 0, r)),
        out_shape=jax.ShapeDtypeStruct((BATCH, 16, NUM_PATCH), jnp.int32),
    )(xn, bias)


def kernel(node_feature, relative_pos):
    b, n, _ = node_feature.shape
    tk_raw = _topk_call(node_feature, relative_pos)
    tk = tk_raw.transpose(0, 2, 1)[:, :, :K]
    src = jnp.broadcast_to(
        jnp.arange(b * n, dtype=jnp.int32).reshape(b, n, 1), (b, n, K))
    edge_list = jnp.stack([tk, src], axis=-1).reshape(-1, 2)
    relation = jnp.zeros((edge_list.shape[0], 1), dtype=edge_list.dtype)
    edge_list = jnp.concatenate([edge_list, relation], axis=-1)
    return (edge_list, 1)


# speedup vs baseline: 1.8761x; 1.0408x over previous
import functools

import jax
import jax.numpy as jnp
from jax import lax
from jax.experimental import pallas as pl
from jax.experimental.pallas import tpu as pltpu

INF = 100000.0
DIM = 96
RES = 32
NUM_PATCH = RES * RES
K = 10
BATCH = 32
RB = 512
NB = NUM_PATCH // RB
NCH = NUM_PATCH // 128

_SORT8 = [
    (0, 1), (2, 3), (4, 5), (6, 7),
    (0, 2), (1, 3), (4, 6), (5, 7),
    (1, 2), (5, 6),
    (0, 4), (1, 5), (2, 6), (3, 7),
    (2, 4), (3, 5),
    (1, 2), (3, 4), (5, 6),
]


def _norm_body(feat_ref, out_ref):
    x = feat_ref[0]
    nrm = jnp.sqrt(jnp.sum(x * x, axis=1, keepdims=True))
    out_ref[0] = x / jnp.clip(nrm, 1e-12, None)


def _bias_body(rel_ref, out_ref):
    r = pl.program_id(0)
    j0 = lax.broadcasted_iota(jnp.int32, (NUM_PATCH, RB), 0)
    i0 = r * RB + lax.broadcasted_iota(jnp.int32, (NUM_PATCH, RB), 1)
    nbr = ((jnp.abs((j0 >> 5) - (i0 >> 5)) <= 1)
           & (jnp.abs((j0 & 31) - (i0 & 31)) <= 1))
    out_ref[...] = rel_ref[0] + jnp.where(nbr, INF, 0.0) + 2.0


def _body(xn_ref, bias_ref, out_ref):
    r = pl.program_id(0)
    b = pl.program_id(1)
    xn = xn_ref[0]
    xr = xn_ref[0, pl.ds(r * RB, RB), :]
    prod = lax.dot_general(xn, xr, (((1,), (1,)), ((), ())),
                           preferred_element_type=jnp.float32)
    d = bias_ref[...] - 2.0 * prod
    ib = lax.bitcast_convert_type(d, jnp.int32)
    ch = []
    for c in range(NCH):
        sl = (ib[c * 128:(c + 1) * 128, :] & jnp.int32(-8)) | jnp.int32(c)
        ch.append(lax.bitcast_convert_type(sl, jnp.float32))
    ch.append(jnp.full((128, RB), jnp.inf, dtype=jnp.float32))

    for a, c in _SORT8:
        lo = jnp.minimum(ch[a], ch[c])
        hi = jnp.maximum(ch[a], ch[c])
        ch[a], ch[c] = lo, hi

    iota0 = lax.broadcasted_iota(jnp.int32, (128, RB), 0)
    outs = []
    for k in range(K):
        v, ix = ch[0], iota0
        for h in (64, 32, 16, 8):
            pred = v[h:] < v[:h]
            v = jnp.where(pred, v[h:], v[:h])
            ix = jnp.where(pred, ix[h:], ix[:h])
        m = jnp.min(v, axis=0)
        jm = jnp.min(jnp.where(v == m[None, :], ix, jnp.int32(1 << 30)),
                     axis=0)
        mc = lax.bitcast_convert_type(m, jnp.int32) & 7
        outs.append(mc * 128 + jm + b * NUM_PATCH)
        upper = min(NCH, K - 1 - k)
        if upper > 0:
            sel = iota0 == jm[None, :]
            for lv in range(upper):
                ch[lv] = jnp.where(sel, ch[lv + 1], ch[lv])
    outs += [jnp.zeros((RB,), jnp.int32)] * (16 - K)
    out_ref[0] = jnp.stack(outs, axis=0)


@functools.partial(jax.jit, static_argnums=())
def _topk_call(node_feature, relative_pos):
    xn = pl.pallas_call(
        _norm_body,
        grid=(BATCH,),
        in_specs=[pl.BlockSpec((1, NUM_PATCH, DIM), lambda b: (b, 0, 0))],
        out_specs=pl.BlockSpec((1, NUM_PATCH, DIM), lambda b: (b, 0, 0)),
        out_shape=jax.ShapeDtypeStruct((BATCH, NUM_PATCH, DIM), jnp.float32),
    )(node_feature)
    bias = pl.pallas_call(
        _bias_body,
        grid=(NB,),
        in_specs=[pl.BlockSpec((1, NUM_PATCH, RB), lambda r: (0, 0, r))],
        out_specs=pl.BlockSpec((NUM_PATCH, RB), lambda r: (0, r)),
        out_shape=jax.ShapeDtypeStruct((NUM_PATCH, NUM_PATCH), jnp.float32),
    )(relative_pos)
    return pl.pallas_call(
        _body,
        grid=(NB, BATCH),
        in_specs=[
            pl.BlockSpec((1, NUM_PATCH, DIM), lambda r, b: (b, 0, 0)),
            pl.BlockSpec((NUM_PATCH, RB), lambda r, b: (0, r)),
        ],
        out_specs=pl.BlockSpec((1, 16, RB), lambda r, b: (b, 0, r)),
        out_shape=jax.ShapeDtypeStruct((BATCH, 16, NUM_PATCH), jnp.int32),
    )(xn, bias)


def kernel(node_feature, relative_pos):
    b, n, _ = node_feature.shape
    tk_raw = _topk_call(node_feature, relative_pos)
    tk = tk_raw.transpose(0, 2, 1)[:, :, :K]
    src = jnp.broadcast_to(
        jnp.arange(b * n, dtype=jnp.int32).reshape(b, n, 1), (b, n, K))
    edge_list = jnp.stack([tk, src], axis=-1).reshape(-1, 2)
    relation = jnp.zeros((edge_list.shape[0], 1), dtype=edge_list.dtype)
    edge_list = jnp.concatenate([edge_list, relation], axis=-1)
    return (edge_list, 1)
